# Initial kernel scaffold; baseline (speedup 1.0000x reference)
import jax
import jax.numpy as jnp
from jax.experimental import pallas as pl


def kernel(hidden_states, router_w, w_gate, w_up, w_down):
    raise NotImplementedError("write your pallas kernel here")


# speedup vs baseline: 1.4342x; 1.4342x over previous
import jax
import jax.numpy as jnp
from jax.experimental import pallas as pl
from jax.experimental.pallas import tpu as pltpu

_E = 8
_H = 768
_I = 1536
_T = 2048


def _moe_dense_body(x_ref, rw_ref, wg_ref, wu_ref, wd_ref, out_ref, logits_ref,
                    comb_ref):
    e = pl.program_id(0)

    @pl.when(e == 0)
    def _router():
        x = x_ref[...]
        logits = jnp.dot(x, rw_ref[...], preferred_element_type=jnp.float32)
        logits_ref[...] = logits
        a = jax.nn.softmax(logits, axis=-1)
        lanes = jax.lax.broadcasted_iota(jnp.int32, (_T, _E), 1)
        m1 = jnp.max(a, axis=-1, keepdims=True)
        i1 = jnp.min(jnp.where(a == m1, lanes, _E), axis=-1, keepdims=True)
        a2 = jnp.where(lanes == i1, -jnp.inf, a)
        m2 = jnp.max(a2, axis=-1, keepdims=True)
        i2 = jnp.min(jnp.where(a2 == m2, lanes, _E), axis=-1, keepdims=True)
        s = m1 + m2
        comb_ref[...] = (jnp.where(lanes == i1, m1 / s, 0.0)
                         + jnp.where(lanes == i2, m2 / s, 0.0))
        out_ref[...] = jnp.zeros_like(out_ref)

    xb = x_ref[...].astype(jnp.bfloat16)
    g = jnp.dot(xb, wg_ref[0], preferred_element_type=jnp.float32)
    u = jnp.dot(xb, wu_ref[0], preferred_element_type=jnp.float32)
    act = (g * jax.nn.sigmoid(g) * u).astype(jnp.bfloat16)
    contrib = jnp.dot(act, wd_ref[0], preferred_element_type=jnp.float32)
    onehot = (jax.lax.broadcasted_iota(jnp.int32, (_E, 1), 0) == e
              ).astype(jnp.float32)
    col = jnp.dot(comb_ref[...], onehot, preferred_element_type=jnp.float32)
    out_ref[...] += contrib * col


def kernel(hidden_states, router_w, w_gate, w_up, w_down):
    shape = hidden_states.shape
    x = hidden_states.reshape(-1, _H)
    wg = w_gate.astype(jnp.bfloat16)
    wu = w_up.astype(jnp.bfloat16)
    wd = w_down.astype(jnp.bfloat16)
    out, logits = pl.pallas_call(
        _moe_dense_body,
        grid=(_E,),
        in_specs=[
            pl.BlockSpec((_T, _H), lambda e: (0, 0)),
            pl.BlockSpec((_H, _E), lambda e: (0, 0)),
            pl.BlockSpec((1, _H, _I), lambda e: (e, 0, 0)),
            pl.BlockSpec((1, _H, _I), lambda e: (e, 0, 0)),
            pl.BlockSpec((1, _I, _H), lambda e: (e, 0, 0)),
        ],
        out_specs=[
            pl.BlockSpec((_T, _H), lambda e: (0, 0)),
            pl.BlockSpec((_T, _E), lambda e: (0, 0)),
        ],
        out_shape=[
            jax.ShapeDtypeStruct((_T, _H), jnp.float32),
            jax.ShapeDtypeStruct((_T, _E), jnp.float32),
        ],
        scratch_shapes=[pltpu.VMEM((_T, _E), jnp.float32)],
    )(x, router_w, wg, wu, wd)
    return out.reshape(shape), logits
